# baseline (speedup 1.0000x reference)
import jax
import jax.numpy as jnp
import numpy as np
from jax import lax
from jax.experimental import pallas as pl
from jax.experimental.pallas import tpu as pltpu
from jax.experimental.pallas import tpu_sc as plsc

_B = 8
_N = 512
_K = 128
_CHUNKS = _N // 16
_UNROLL = 4


def _threefry2x32(k0, k1, x0, x1):
    ks2 = k0 ^ k1 ^ np.uint32(0x1BD11BDA)
    ks = (k0, k1, ks2)
    rots = ((13, 15, 26, 6), (17, 29, 16, 24))
    x0 = x0 + ks[0]
    x1 = x1 + ks[1]
    for i in range(5):
        for r in rots[i % 2]:
            x0 = x0 + x1
            x1 = (x1 << np.uint32(r)) | (x1 >> np.uint32(32 - r))
            x1 = x0 ^ x1
        x0 = x0 + ks[(i + 1) % 3]
        x1 = x1 + ks[(i + 2) % 3] + np.uint32(i + 1)
    return x0, x1


def _splat(x, dtype=jnp.int32):
    return jnp.full((16,), x, dtype)


def _popcount(mask):
    return plsc.all_reduce_population_count(mask)


def _sc_body(out_hbm, r_buf, hist, hist16, cand, out_row, sem):
    wid = lax.axis_index("s")

    @pl.when(wid < _B)
    def _work():
        b = wid
        lanes = lax.iota(jnp.int32, 16)
        zeros_u = _splat(0, jnp.uint32)

        kb0, kb1 = _threefry2x32(zeros_u, _splat(42, jnp.uint32), zeros_u,
                                 _splat(b, jnp.uint32).astype(jnp.uint32))
        sk0, sk1 = _threefry2x32(kb0, kb1, zeros_u, _splat(1, jnp.uint32))

        for c in range(256 // 16):
            hist[pl.ds(c * 16, 16)] = _splat(0, jnp.int32)
        hist16[...] = _splat(0, jnp.int32)

        ones = _splat(1, jnp.int32)

        for c in range(_CHUNKS):
            ctr = (lanes + c * 16).astype(jnp.uint32)
            y0, y1 = _threefry2x32(sk0, sk1, zeros_u, ctr)
            r = y0 ^ y1
            s = lax.bitcast_convert_type(r ^ np.uint32(0x80000000),
                                         jnp.int32)
            r_buf[pl.ds(c * 16, 16)] = s
            plsc.addupdate_scatter(
                hist, [(r >> np.uint32(24)).astype(jnp.int32)], ones)
            plsc.addupdate_scatter(
                hist16, [(r >> np.uint32(28)).astype(jnp.int32)], ones)

        h16 = hist16[...]
        cum16 = plsc.cumsum(h16)
        g = jnp.max(plsc.all_reduce_ffs(cum16 >= _K))
        gs = _splat(g)
        below_g = jnp.max(jnp.where(lanes == gs, cum16 - h16, 0))
        h = hist[pl.ds(g * 16, 16)]
        cum = plsc.cumsum(h) + _splat(below_g)
        t_off = jnp.max(plsc.all_reduce_ffs(cum >= _K))
        ts = _splat(t_off)
        t_bucket = g * 16 + t_off
        below = jnp.max(jnp.where(lanes == ts, cum - h, 0))

        cand[...] = _splat(0x7FFFFFFF, jnp.int32)
        tb_splat = _splat(t_bucket)

        acc = _splat(0, jnp.int32)
        for c in range(_CHUNKS):
            s = r_buf[pl.ds(c * 16, 16)]
            r = (lax.bitcast_convert_type(s, jnp.uint32)
                 ^ np.uint32(0x80000000))
            mask = (r >> np.uint32(24)).astype(jnp.int32) == tb_splat
            mi = mask.astype(jnp.int32)
            pos = plsc.cumsum(mi) - 1 + acc
            plsc.store_scatter(cand, [pos], s, mask=mask)
            acc = acc + _popcount(mask)

        candv = cand[...]
        srt = plsc.sort_key_val(candv, candv)
        cand_sorted = srt[0] if isinstance(srt, (tuple, list)) else srt
        v = jnp.max(jnp.where(lanes == _splat(_K - 1) - _splat(below),
                              cand_sorted, _splat(-0x80000000)))
        v_splat = _splat(v, jnp.int32)

        off = _splat(0, jnp.int32)
        for c in range(_CHUNKS):
            s = r_buf[pl.ds(c * 16, 16)]
            mask = s <= v_splat
            mi = mask.astype(jnp.int32)
            pos = plsc.cumsum(mi) - 1 + off
            plsc.store_scatter(out_row, [pos], lanes + c * 16, mask=mask)
            off = off + _popcount(mask)

        pltpu.sync_copy(out_row, out_hbm.at[b])


def kernel(images, features):
    del images, features
    sampler = pl.kernel(
        _sc_body,
        out_type=jax.ShapeDtypeStruct((_B, _K), jnp.int32),
        mesh=plsc.VectorSubcoreMesh(core_axis_name="c", subcore_axis_name="s",
                                    num_cores=1),
        compiler_params=pltpu.CompilerParams(needs_layout_passes=False),
        scratch_types=[
            pltpu.VMEM((_N,), jnp.int32),
            pltpu.VMEM((256,), jnp.int32),
            pltpu.VMEM((16,), jnp.int32),
            pltpu.VMEM((16,), jnp.int32),
            pltpu.VMEM((_K,), jnp.int32),
            pltpu.SemaphoreType.DMA,
        ],
    )
    return sampler()


# speedup vs baseline: 1.1985x; 1.1985x over previous
import jax
import jax.numpy as jnp
import numpy as np
from jax import lax
from jax.experimental import pallas as pl
from jax.experimental.pallas import tpu as pltpu
from jax.experimental.pallas import tpu_sc as plsc

_B = 8
_N = 512
_K = 128
_CHUNKS = _N // 16


def _threefry2x32(k0, k1, x0, x1):
    ks2 = k0 ^ k1 ^ np.uint32(0x1BD11BDA)
    ks = (k0, k1, ks2)
    rots = ((13, 15, 26, 6), (17, 29, 16, 24))
    x0 = x0 + ks[0]
    x1 = x1 + ks[1]
    for i in range(5):
        for r in rots[i % 2]:
            x0 = x0 + x1
            x1 = (x1 << np.uint32(r)) | (x1 >> np.uint32(32 - r))
            x1 = x0 ^ x1
        x0 = x0 + ks[(i + 1) % 3]
        x1 = x1 + ks[(i + 2) % 3] + np.uint32(i + 1)
    return x0, x1


def _splat(x, dtype=jnp.int32):
    return jnp.full((16,), x, dtype)


def _popcount(mask):
    return plsc.all_reduce_population_count(mask)


def _sc_body(out_hbm, r_buf, hist, hist16, cand, out_row, sem):
    wid = lax.axis_index("s")

    @pl.when(wid < _B)
    def _work():
        b = wid
        lanes = lax.iota(jnp.int32, 16)
        zeros_u = _splat(0, jnp.uint32)
        ones = _splat(1, jnp.int32)

        def _kc(i, carry):
            k0, k1, x1 = carry
            y0, y1 = _threefry2x32(k0, k1, zeros_u, x1)
            return y0, y1, _splat(1, jnp.uint32)
        sk0, sk1, _ = lax.fori_loop(
            0, 2, _kc,
            (zeros_u, _splat(42, jnp.uint32),
             _splat(b, jnp.uint32).astype(jnp.uint32)))

        def _z(c, carry):
            hist[pl.ds(c * 16, 16)] = _splat(0, jnp.int32)
            return carry
        lax.fori_loop(0, 256 // 16, _z, 0)
        hist16[...] = _splat(0, jnp.int32)

        def _gen(c, carry):
            ctr = (lanes + c * 16).astype(jnp.uint32)
            y0, y1 = _threefry2x32(sk0, sk1, zeros_u, ctr)
            r = y0 ^ y1
            s = lax.bitcast_convert_type(r ^ np.uint32(0x80000000), jnp.int32)
            r_buf[pl.ds(c * 16, 16)] = s
            plsc.addupdate_scatter(
                hist, [(r >> np.uint32(24)).astype(jnp.int32)], ones)
            plsc.addupdate_scatter(
                hist16, [(r >> np.uint32(28)).astype(jnp.int32)], ones)
            return carry
        lax.fori_loop(0, _CHUNKS, _gen, 0)

        h16 = hist16[...]
        cum16 = plsc.cumsum(h16)
        g = jnp.max(plsc.all_reduce_ffs(cum16 >= _K))
        gs = _splat(g)
        below_g = jnp.max(jnp.where(lanes == gs, cum16 - h16, 0))
        h = hist[pl.ds(g * 16, 16)]
        cum = plsc.cumsum(h) + _splat(below_g)
        t_off = jnp.max(plsc.all_reduce_ffs(cum >= _K))
        ts = _splat(t_off)
        t_bucket = g * 16 + t_off
        below = jnp.max(jnp.where(lanes == ts, cum - h, 0))

        cand[...] = _splat(0x7FFFFFFF, jnp.int32)
        tb_splat = _splat(t_bucket)

        def _cand(c, coff):
            s = r_buf[pl.ds(c * 16, 16)]
            r = (lax.bitcast_convert_type(s, jnp.uint32)
                 ^ np.uint32(0x80000000))
            mask = (r >> np.uint32(24)).astype(jnp.int32) == tb_splat
            mi = mask.astype(jnp.int32)
            pos = plsc.cumsum(mi) - 1 + coff
            plsc.store_scatter(cand, [pos], s, mask=mask)
            return coff + _popcount(mask)
        lax.fori_loop(0, _CHUNKS, _cand, _splat(0, jnp.int32))

        candv = cand[...]
        srt = plsc.sort_key_val(candv, candv)
        cand_sorted = srt[0] if isinstance(srt, (tuple, list)) else srt
        v = jnp.max(jnp.where(lanes == _splat(_K - 1) - _splat(below),
                              cand_sorted, _splat(-0x80000000)))
        v_splat = _splat(v, jnp.int32)

        def _emit(c, off):
            s = r_buf[pl.ds(c * 16, 16)]
            mask = s <= v_splat
            mi = mask.astype(jnp.int32)
            pos = plsc.cumsum(mi) - 1 + off
            plsc.store_scatter(out_row, [pos], lanes + c * 16, mask=mask)
            return off + _popcount(mask)
        lax.fori_loop(0, _CHUNKS, _emit, _splat(0, jnp.int32))

        pltpu.sync_copy(out_row, out_hbm.at[b])


def kernel(images, features):
    del images, features
    sampler = pl.kernel(
        _sc_body,
        out_type=jax.ShapeDtypeStruct((_B, _K), jnp.int32),
        mesh=plsc.VectorSubcoreMesh(core_axis_name="c", subcore_axis_name="s",
                                    num_cores=1),
        compiler_params=pltpu.CompilerParams(needs_layout_passes=False),
        scratch_types=[
            pltpu.VMEM((_N,), jnp.int32),
            pltpu.VMEM((256,), jnp.int32),
            pltpu.VMEM((16,), jnp.int32),
            pltpu.VMEM((16,), jnp.int32),
            pltpu.VMEM((_K,), jnp.int32),
            pltpu.SemaphoreType.DMA,
        ],
    )
    return sampler()
